# baseline (speedup 1.0000x reference)
import functools

import jax
import jax.numpy as jnp
from jax.experimental import pallas as pl
from jax.experimental.pallas import tpu as pltpu

_TARGET_BLOCK_BYTES = 4 * 1024 * 1024


def _largest_divisor_at_most(n, cap):
    cap = max(1, min(n, cap))
    for t in range(cap, 0, -1):
        if n % t == 0:
            return t
    return 1


def _poolmean_kernel(x_ref, o_ref, *, inv_s):
    s = jnp.sum(x_ref[...], axis=1, dtype=jnp.float32)
    o_ref[...] = (s * inv_s).astype(o_ref.dtype)


def kernel(x):
    n, c, d, h, w = x.shape
    s = d * h * w
    xt = jnp.transpose(x.reshape(n, c, s), (0, 2, 1))
    itemsize = xt.dtype.itemsize

    per_sample = s * c * itemsize
    tb_cap = max(1, _TARGET_BLOCK_BYTES // per_sample)
    if n >= 2:
        tb_cap = min(tb_cap, n // 2)
    tb = _largest_divisor_at_most(n, tb_cap)
    nb = n // tb

    cost = pl.CostEstimate(
        flops=n * c * s,
        transcendentals=0,
        bytes_accessed=n * c * s * itemsize + n * c * itemsize,
    )

    out = pl.pallas_call(
        functools.partial(_poolmean_kernel, inv_s=1.0 / s),
        out_shape=jax.ShapeDtypeStruct((n, c), xt.dtype),
        grid_spec=pltpu.PrefetchScalarGridSpec(
            num_scalar_prefetch=0,
            grid=(nb,),
            in_specs=[pl.BlockSpec((tb, s, c), lambda i: (i, 0, 0))],
            out_specs=pl.BlockSpec((tb, c), lambda i: (i, 0)),
        ),
        compiler_params=pltpu.CompilerParams(
            dimension_semantics=("parallel",),
        ),
        cost_estimate=cost,
    )(xt)

    return out.reshape(-1, 512)


# speedup vs baseline: 1.1036x; 1.1036x over previous
import functools

import jax
import jax.numpy as jnp
from jax.experimental import pallas as pl
from jax.experimental.pallas import tpu as pltpu

_TARGET_BLOCK_BYTES = 8 * 1024 * 1024


def _largest_divisor_at_most(n, cap):
    cap = max(1, min(n, cap))
    for t in range(cap, 0, -1):
        if n % t == 0:
            return t
    return 1


def _poolmean_kernel(x_ref, o_ref, *, inv_s):
    s = jnp.sum(x_ref[...], axis=1, keepdims=True, dtype=jnp.float32)
    o_ref[...] = (s * inv_s).astype(o_ref.dtype)


def kernel(x):
    n, c, d, h, w = x.shape
    s = d * h * w
    xt = jnp.transpose(x.reshape(n, c, s), (0, 2, 1))
    itemsize = xt.dtype.itemsize

    per_sample = s * c * itemsize
    tb_cap = max(1, _TARGET_BLOCK_BYTES // per_sample)
    if n >= 2:
        tb_cap = min(tb_cap, n // 2)
    tb = _largest_divisor_at_most(n, tb_cap)
    nb = n // tb

    cost = pl.CostEstimate(
        flops=n * c * s,
        transcendentals=0,
        bytes_accessed=n * c * s * itemsize + n * c * itemsize,
    )

    out = pl.pallas_call(
        functools.partial(_poolmean_kernel, inv_s=1.0 / s),
        out_shape=jax.ShapeDtypeStruct((n, 1, c), xt.dtype),
        grid_spec=pltpu.PrefetchScalarGridSpec(
            num_scalar_prefetch=0,
            grid=(nb,),
            in_specs=[pl.BlockSpec((tb, s, c), lambda i: (i, 0, 0))],
            out_specs=pl.BlockSpec((tb, 1, c), lambda i: (i, 0, 0)),
        ),
        compiler_params=pltpu.CompilerParams(
            dimension_semantics=("parallel",),
        ),
        cost_estimate=cost,
    )(xt)

    return out.reshape(-1, 512)
